# Initial kernel scaffold; baseline (speedup 1.0000x reference)
import jax
import jax.numpy as jnp
from jax.experimental import pallas as pl


def kernel(x, params, adj):
    raise NotImplementedError("write your pallas kernel here")


# speedup vs baseline: 51.5216x; 51.5216x over previous
import functools

import jax
import jax.numpy as jnp
from jax import lax
from jax.experimental import pallas as pl
from jax.experimental.pallas import tpu as pltpu
from jax.experimental.pallas import tpu_sc as plsc

N = 10000
E = 320000
N_PAD = 10240
NC = 2
NS = 16
NW = NC * NS
CHUNK = 128
CPW = 80
E_PAD = NW * CPW * CHUNK
ROWS_PER_TILE = N_PAD // NS

ALPHA = 0.9
EPS_BN = 0.001
QEXP = (ALPHA + 1.0) / 2.0

BLK = 2048


def _sc_degree(dst_idx):
    mesh = plsc.VectorSubcoreMesh(
        core_axis_name="c", subcore_axis_name="s", num_cores=NC, num_subcores=NS)

    @functools.partial(
        pl.kernel,
        out_type=jax.ShapeDtypeStruct((NC, N_PAD), jnp.float32),
        mesh=mesh,
        compiler_params=pltpu.CompilerParams(use_tc_tiling_on_sc=False),
        scratch_types=[
            pltpu.VMEM((CPW, CHUNK), jnp.int32),
            pltpu.VMEM((CHUNK,), jnp.float32),
            pltpu.VMEM_SHARED((N_PAD,), jnp.float32),
        ],
    )
    def k(dst_hbm, out_hbm, idx_v, buf_v, acc):
        c = lax.axis_index("c")
        s = lax.axis_index("s")
        w = s * NC + c
        pltpu.sync_copy(dst_hbm.at[w], idx_v)
        for i in range(CHUNK // 16):
            buf_v[pl.ds(i * 16, 16)] = jnp.zeros((16,), jnp.float32)
        for kk in range(ROWS_PER_TILE // CHUNK):
            pltpu.sync_copy(
                buf_v, acc.at[pl.ds(s * ROWS_PER_TILE + kk * CHUNK, CHUNK)])
        for i in range(CHUNK // 16):
            buf_v[pl.ds(i * 16, 16)] = jnp.ones((16,), jnp.float32)
        plsc.subcore_barrier()

        def body(j, carry):
            pltpu.sync_copy(buf_v, acc.at[idx_v.at[j]], add=True)
            return carry

        lax.fori_loop(0, CPW, body, 0)
        plsc.subcore_barrier()
        pltpu.sync_copy(
            acc.at[pl.ds(s * ROWS_PER_TILE, ROWS_PER_TILE)],
            out_hbm.at[c, pl.ds(s * ROWS_PER_TILE, ROWS_PER_TILE)])

    return k(dst_idx)


def _sc_propagate(xs, src_idx, dst_idx, f):
    mesh = plsc.VectorSubcoreMesh(
        core_axis_name="c", subcore_axis_name="s", num_cores=NC, num_subcores=NS)

    @functools.partial(
        pl.kernel,
        out_type=jax.ShapeDtypeStruct((NC, N_PAD, f), jnp.float32),
        mesh=mesh,
        compiler_params=pltpu.CompilerParams(use_tc_tiling_on_sc=False),
        scratch_types=[
            pltpu.VMEM((CPW, CHUNK), jnp.int32),
            pltpu.VMEM((CPW, CHUNK), jnp.int32),
            pltpu.VMEM((2, CHUNK, f), jnp.float32),
            pltpu.VMEM_SHARED((N_PAD, f), jnp.float32),
            pltpu.SemaphoreType.DMA,
            pltpu.SemaphoreType.DMA,
        ],
    )
    def k(xs_hbm, src_hbm, dst_hbm, out_hbm, src_v, dst_v, rows_v, acc,
          sem0, sem1):
        c = lax.axis_index("c")
        s = lax.axis_index("s")
        w = s * NC + c
        pltpu.sync_copy(src_hbm.at[w], src_v)
        pltpu.sync_copy(dst_hbm.at[w], dst_v)

        zero16 = jnp.zeros((16,), jnp.float32)

        def zrow(r, carry):
            for i in range(f // 16):
                rows_v[0, r, pl.ds(i * 16, 16)] = zero16
            return carry

        lax.fori_loop(0, CHUNK, zrow, 0)
        for kk in range(ROWS_PER_TILE // CHUNK):
            pltpu.sync_copy(
                rows_v.at[0],
                acc.at[pl.ds(s * ROWS_PER_TILE + kk * CHUNK, CHUNK)])
        plsc.subcore_barrier()

        pltpu.async_copy(xs_hbm.at[src_v.at[0]], rows_v.at[0], sem0)
        pltpu.async_copy(xs_hbm.at[src_v.at[1]], rows_v.at[1], sem1)

        def body(jo, carry):
            j0 = 2 * jo
            pltpu.make_async_copy(
                xs_hbm.at[src_v.at[j0]], rows_v.at[0], sem0).wait()
            pltpu.sync_copy(rows_v.at[0], acc.at[dst_v.at[j0]], add=True)
            pltpu.async_copy(xs_hbm.at[src_v.at[j0 + 2]], rows_v.at[0], sem0)
            j1 = j0 + 1
            pltpu.make_async_copy(
                xs_hbm.at[src_v.at[j1]], rows_v.at[1], sem1).wait()
            pltpu.sync_copy(rows_v.at[1], acc.at[dst_v.at[j1]], add=True)
            pltpu.async_copy(xs_hbm.at[src_v.at[j1 + 2]], rows_v.at[1], sem1)
            return carry

        lax.fori_loop(0, CPW // 2 - 1, body, 0)
        jt = CPW - 2
        pltpu.make_async_copy(
            xs_hbm.at[src_v.at[jt]], rows_v.at[0], sem0).wait()
        pltpu.sync_copy(rows_v.at[0], acc.at[dst_v.at[jt]], add=True)
        pltpu.make_async_copy(
            xs_hbm.at[src_v.at[jt + 1]], rows_v.at[1], sem1).wait()
        pltpu.sync_copy(rows_v.at[1], acc.at[dst_v.at[jt + 1]], add=True)

        plsc.subcore_barrier()
        pltpu.sync_copy(
            acc.at[pl.ds(s * ROWS_PER_TILE, ROWS_PER_TILE)],
            out_hbm.at[c].at[pl.ds(s * ROWS_PER_TILE, ROWS_PER_TILE)])

    return k(xs, src_idx, dst_idx)


def _elu(v):
    return jnp.where(v > 0.0, v, jnp.exp(v) - 1.0)


def _enc_body(x_ref, degt_ref, w0_ref, b0_ref, w1_ref, b1_ref,
              feat_ref, xs1_ref, dinv_ref):
    h = _elu(jnp.dot(x_ref[...], w0_ref[...],
                     preferred_element_type=jnp.float32) + b0_ref[...])
    feat = _elu(jnp.dot(h, w1_ref[...],
                        preferred_element_type=jnp.float32) + b1_ref[...])
    d = degt_ref[...]
    degsum = d[:, 0:1] + d[:, 1:2] + 1.0
    dinv = lax.rsqrt(degsum)
    feat_ref[...] = feat
    xs1_ref[...] = feat * dinv
    dinv_ref[...] = dinv


def _tc_encoder(x_pad, deg_t, w0, b0, w1, b1):
    grid = (N_PAD // BLK,)
    return pl.pallas_call(
        _enc_body,
        grid=grid,
        in_specs=[
            pl.BlockSpec((BLK, 128), lambda i: (i, 0)),
            pl.BlockSpec((BLK, NC), lambda i: (i, 0)),
            pl.BlockSpec((128, 32), lambda i: (0, 0)),
            pl.BlockSpec((1, 32), lambda i: (0, 0)),
            pl.BlockSpec((32, 32), lambda i: (0, 0)),
            pl.BlockSpec((1, 32), lambda i: (0, 0)),
        ],
        out_specs=[
            pl.BlockSpec((BLK, 32), lambda i: (i, 0)),
            pl.BlockSpec((BLK, 32), lambda i: (i, 0)),
            pl.BlockSpec((BLK, 1), lambda i: (i, 0)),
        ],
        out_shape=[
            jax.ShapeDtypeStruct((N_PAD, 32), jnp.float32),
            jax.ShapeDtypeStruct((N_PAD, 32), jnp.float32),
            jax.ShapeDtypeStruct((N_PAD, 1), jnp.float32),
        ],
    )(x_pad, deg_t, w0, b0, w1, b1)


def _mid_body(p1_ref, xs1_ref, dinv_ref, wc_ref, bc_ref, wcat_ref, xs2_ref):
    a = p1_ref[...]
    pf = (a[0] + a[1] + xs1_ref[...]) * dinv_ref[...]
    cc = jnp.maximum(
        jnp.dot(pf, wc_ref[...], preferred_element_type=jnp.float32)
        + bc_ref[...], 0.0)
    t = jnp.dot(cc, wcat_ref[...], preferred_element_type=jnp.float32)
    xs2_ref[...] = t * dinv_ref[...]


def _tc_mid(p1, xs1, dinv, wc, bc, wcat):
    grid = (N_PAD // BLK,)
    return pl.pallas_call(
        _mid_body,
        grid=grid,
        in_specs=[
            pl.BlockSpec((NC, BLK, 32), lambda i: (0, i, 0)),
            pl.BlockSpec((BLK, 32), lambda i: (i, 0)),
            pl.BlockSpec((BLK, 1), lambda i: (i, 0)),
            pl.BlockSpec((32, 64), lambda i: (0, 0)),
            pl.BlockSpec((1, 64), lambda i: (0, 0)),
            pl.BlockSpec((64, 16), lambda i: (0, 0)),
        ],
        out_specs=pl.BlockSpec((BLK, 16), lambda i: (i, 0)),
        out_shape=jax.ShapeDtypeStruct((N_PAD, 16), jnp.float32),
    )(p1, xs1, dinv, wc, bc, wcat)


def _tail_body(p2_ref, xs2_ref, dinv_ref, feat_ref, bcat_ref,
               wd0a_ref, wd0b_ref, bd0_ref, wd1_ref, bd1_ref,
               cla_ref, clb_ref, csq_ref,
               mu_ref, lv_ref, de_ref, q_ref):
    a = p2_ref[...]
    pt = (a[0] + a[1] + xs2_ref[...]) * dinv_ref[...] + bcat_ref[...]
    mu = pt[:, 0:8]
    feat = feat_ref[...]
    dh = _elu(
        jnp.dot(feat, wd0a_ref[...], preferred_element_type=jnp.float32)
        + jnp.dot(mu, wd0b_ref[...], preferred_element_type=jnp.float32)
        + bd0_ref[...])
    de = jnp.dot(dh, wd1_ref[...],
                 preferred_element_type=jnp.float32) + bd1_ref[...]
    zsq = (jnp.sum(feat * feat, axis=1, keepdims=True)
           + jnp.sum(mu * mu, axis=1, keepdims=True))
    zc = (jnp.dot(feat, cla_ref[...], preferred_element_type=jnp.float32)
          + jnp.dot(mu, clb_ref[...], preferred_element_type=jnp.float32))
    qd = zsq - 2.0 * zc + csq_ref[...]
    u = 1.0 / (1.0 + qd * (1.0 / ALPHA))
    qq = jnp.exp(QEXP * jnp.log(u))
    mu_ref[...] = mu
    lv_ref[...] = pt[:, 8:16]
    de_ref[...] = de
    q_ref[...] = qq / jnp.sum(qq, axis=1, keepdims=True)


def _tc_tail(p2, xs2, dinv, feat, bcat, wd0a, wd0b, bd0, wd1, bd1,
             cla, clb, csq):
    grid = (N_PAD // BLK,)
    return pl.pallas_call(
        _tail_body,
        grid=grid,
        in_specs=[
            pl.BlockSpec((NC, BLK, 16), lambda i: (0, i, 0)),
            pl.BlockSpec((BLK, 16), lambda i: (i, 0)),
            pl.BlockSpec((BLK, 1), lambda i: (i, 0)),
            pl.BlockSpec((BLK, 32), lambda i: (i, 0)),
            pl.BlockSpec((1, 16), lambda i: (0, 0)),
            pl.BlockSpec((32, 32), lambda i: (0, 0)),
            pl.BlockSpec((8, 32), lambda i: (0, 0)),
            pl.BlockSpec((1, 32), lambda i: (0, 0)),
            pl.BlockSpec((32, 128), lambda i: (0, 0)),
            pl.BlockSpec((1, 128), lambda i: (0, 0)),
            pl.BlockSpec((32, 15), lambda i: (0, 0)),
            pl.BlockSpec((8, 15), lambda i: (0, 0)),
            pl.BlockSpec((1, 15), lambda i: (0, 0)),
        ],
        out_specs=[
            pl.BlockSpec((BLK, 8), lambda i: (i, 0)),
            pl.BlockSpec((BLK, 8), lambda i: (i, 0)),
            pl.BlockSpec((BLK, 128), lambda i: (i, 0)),
            pl.BlockSpec((BLK, 15), lambda i: (i, 0)),
        ],
        out_shape=[
            jax.ShapeDtypeStruct((N_PAD, 8), jnp.float32),
            jax.ShapeDtypeStruct((N_PAD, 8), jnp.float32),
            jax.ShapeDtypeStruct((N_PAD, 128), jnp.float32),
            jax.ShapeDtypeStruct((N_PAD, 15), jnp.float32),
        ],
    )(p2, xs2, dinv, feat, bcat, wd0a, wd0b, bd0, wd1, bd1, cla, clb, csq)


def kernel(x, params, adj):
    p = params
    s_bn = (1.0 + EPS_BN) ** -0.5

    w0 = p["enc_W0"] * (p["enc_g0"] * s_bn)[None, :]
    b0 = (p["enc_b0"] * p["enc_g0"] * s_bn + p["enc_be0"])[None, :]
    w1 = p["enc_W1"] * (p["enc_g1"] * s_bn)[None, :]
    b1 = (p["enc_b1"] * p["enc_g1"] * s_bn + p["enc_be1"])[None, :]
    w1p = jnp.pad(w1, ((0, 0), (0, 12)))
    b1p = jnp.pad(b1, ((0, 0), (0, 12)))
    wc = p["conv_W"] * (p["conv_g"] * s_bn)[None, :]
    bc = (p["conv_b"] * p["conv_g"] * s_bn + p["conv_be"])[None, :]
    wcp = jnp.pad(wc, ((0, 12), (0, 0)))
    wcat = jnp.concatenate([p["mean_W"], p["logv_W"]], axis=1)
    bcat = jnp.concatenate([p["mean_b"], p["logv_b"]])[None, :]
    wd0 = p["dec_W0"] * (p["dec_g0"] * s_bn)[None, :]
    bd0 = (p["dec_b0"] * p["dec_g0"] * s_bn + p["dec_be0"])[None, :]
    wd0a = jnp.pad(wd0[:20], ((0, 12), (0, 0)))
    wd0b = wd0[20:]
    wd1 = p["dec_W1"]
    bd1 = p["dec_b1"][None, :]
    cl = p["cluster"]
    cla = jnp.pad(cl[:, :20].T, ((0, 12), (0, 0)))
    clb = cl[:, 20:].T
    csq = jnp.sum(cl * cl, axis=1)[None, :]

    src = adj[0]
    dst = adj[1]
    ar = jnp.arange(E_PAD - E, dtype=jnp.int32)
    src_pad = jnp.concatenate([src, ar % 997]).reshape(NW, CPW, CHUNK)
    dst_pad = jnp.concatenate([dst, N + ar % (N_PAD - N)]).reshape(NW, CPW, CHUNK)

    x_pad = jnp.pad(x, ((0, N_PAD - N), (0, 0)))

    deg = _sc_degree(dst_pad)
    feat, xs1, dinv = _tc_encoder(x_pad, deg.T, w0, b0, w1p, b1p)
    p1 = _sc_propagate(xs1, src_pad, dst_pad, 32)
    xs2 = _tc_mid(p1, xs1, dinv, wcp, bc, wcat)
    p2 = _sc_propagate(xs2, src_pad, dst_pad, 16)
    mu, logvar, de_feat, q = _tc_tail(
        p2, xs2, dinv, feat, bcat, wd0a, wd0b, bd0, wd1, bd1, cla, clb, csq)

    mu_n = mu[:N]
    feat_n = feat[:N, :20]
    z = jnp.concatenate([feat_n, mu_n], axis=1)
    return (z, mu_n, logvar[:N], de_feat[:N], q[:N], feat_n, mu_n)


# speedup vs baseline: 59.5549x; 1.1559x over previous
import functools

import jax
import jax.numpy as jnp
from jax import lax
from jax.experimental import pallas as pl
from jax.experimental.pallas import tpu as pltpu
from jax.experimental.pallas import tpu_sc as plsc

N = 10000
E = 320000
N_PAD = 10240
NC = 2
NS = 16
NW = NC * NS
CHUNK = 128
CPW = 80
E_PAD = NW * CPW * CHUNK
ROWS_PER_TILE = N_PAD // NS

ALPHA = 0.9
EPS_BN = 0.001
QEXP = (ALPHA + 1.0) / 2.0

BLK = 2048


def _sc_degree(dst_idx):
    mesh = plsc.VectorSubcoreMesh(
        core_axis_name="c", subcore_axis_name="s", num_cores=NC, num_subcores=NS)

    @functools.partial(
        pl.kernel,
        out_type=jax.ShapeDtypeStruct((NC, N_PAD), jnp.float32),
        mesh=mesh,
        compiler_params=pltpu.CompilerParams(use_tc_tiling_on_sc=False),
        scratch_types=[
            pltpu.VMEM((CPW, CHUNK), jnp.int32),
            pltpu.VMEM((CHUNK,), jnp.float32),
            pltpu.VMEM_SHARED((N_PAD,), jnp.float32),
            pltpu.SemaphoreType.DMA,
        ],
    )
    def k(dst_hbm, out_hbm, idx_v, buf_v, acc, sem):
        c = lax.axis_index("c")
        s = lax.axis_index("s")
        w = s * NC + c
        pltpu.sync_copy(dst_hbm.at[w], idx_v)
        for i in range(CHUNK // 16):
            buf_v[pl.ds(i * 16, 16)] = jnp.zeros((16,), jnp.float32)
        for kk in range(ROWS_PER_TILE // CHUNK):
            pltpu.sync_copy(
                buf_v, acc.at[pl.ds(s * ROWS_PER_TILE + kk * CHUNK, CHUNK)])
        for i in range(CHUNK // 16):
            buf_v[pl.ds(i * 16, 16)] = jnp.ones((16,), jnp.float32)
        plsc.subcore_barrier()

        grp = 16

        def body(g, carry):
            for i in range(grp):
                pltpu.async_copy(buf_v, acc.at[idx_v.at[g * grp + i]], sem,
                                 add=True)
            for i in range(grp):
                pltpu.make_async_copy(
                    buf_v, acc.at[idx_v.at[g * grp + i]], sem).wait()
            return carry

        lax.fori_loop(0, CPW // grp, body, 0)
        plsc.subcore_barrier()
        pltpu.sync_copy(
            acc.at[pl.ds(s * ROWS_PER_TILE, ROWS_PER_TILE)],
            out_hbm.at[c, pl.ds(s * ROWS_PER_TILE, ROWS_PER_TILE)])

    return k(dst_idx)


def _sc_propagate(xs, src_idx, dst_idx, f):
    mesh = plsc.VectorSubcoreMesh(
        core_axis_name="c", subcore_axis_name="s", num_cores=NC, num_subcores=NS)

    @functools.partial(
        pl.kernel,
        out_type=jax.ShapeDtypeStruct((NC, N_PAD, f), jnp.float32),
        mesh=mesh,
        compiler_params=pltpu.CompilerParams(use_tc_tiling_on_sc=False),
        scratch_types=[
            pltpu.VMEM((CPW, CHUNK), jnp.int32),
            pltpu.VMEM((CPW, CHUNK), jnp.int32),
            pltpu.VMEM((4, CHUNK, f), jnp.float32),
            pltpu.VMEM_SHARED((N_PAD, f), jnp.float32),
            [pltpu.SemaphoreType.DMA] * 4,
            [pltpu.SemaphoreType.DMA] * 4,
        ],
    )
    def k(xs_hbm, src_hbm, dst_hbm, out_hbm, src_v, dst_v, rows_v, acc,
          gsem, ssem):
        c = lax.axis_index("c")
        s = lax.axis_index("s")
        w = s * NC + c
        pltpu.sync_copy(src_hbm.at[w], src_v)
        pltpu.sync_copy(dst_hbm.at[w], dst_v)

        zero16 = jnp.zeros((16,), jnp.float32)

        def zrow(r, carry):
            for i in range(f // 16):
                rows_v[0, r, pl.ds(i * 16, 16)] = zero16
            return carry

        lax.fori_loop(0, CHUNK, zrow, 0)
        for kk in range(ROWS_PER_TILE // CHUNK):
            pltpu.sync_copy(
                rows_v.at[0],
                acc.at[pl.ds(s * ROWS_PER_TILE + kk * CHUNK, CHUNK)])
        plsc.subcore_barrier()

        for b in range(4):
            pltpu.async_copy(xs_hbm.at[src_v.at[b]], rows_v.at[b], gsem[b])

        def body(jo, carry):
            j0 = 4 * jo
            for b in range(4):
                j = j0 + b
                pltpu.make_async_copy(
                    xs_hbm.at[src_v.at[j]], rows_v.at[b], gsem[b]).wait()
                pltpu.async_copy(rows_v.at[b], acc.at[dst_v.at[j]], ssem[b],
                                 add=True)
            for b in range(4):
                j = j0 + b

                @pl.when(j + 4 < CPW)
                def _():
                    pltpu.make_async_copy(
                        rows_v.at[b], acc.at[dst_v.at[j]], ssem[b]).wait()
                    pltpu.async_copy(
                        xs_hbm.at[src_v.at[j + 4]], rows_v.at[b], gsem[b])
            return carry

        lax.fori_loop(0, CPW // 4, body, 0)
        for b in range(4):
            j = CPW - 4 + b
            pltpu.make_async_copy(
                rows_v.at[b], acc.at[dst_v.at[j]], ssem[b]).wait()

        plsc.subcore_barrier()
        pltpu.sync_copy(
            acc.at[pl.ds(s * ROWS_PER_TILE, ROWS_PER_TILE)],
            out_hbm.at[c].at[pl.ds(s * ROWS_PER_TILE, ROWS_PER_TILE)])

    return k(xs, src_idx, dst_idx)


def _elu(v):
    return jnp.where(v > 0.0, v, jnp.exp(v) - 1.0)


def _enc_body(x_ref, degt_ref, w0_ref, b0_ref, w1_ref, b1_ref,
              feat_ref, xs1_ref, dinv_ref):
    h = _elu(jnp.dot(x_ref[...], w0_ref[...],
                     preferred_element_type=jnp.float32) + b0_ref[...])
    feat = _elu(jnp.dot(h, w1_ref[...],
                        preferred_element_type=jnp.float32) + b1_ref[...])
    d = degt_ref[...]
    degsum = d[:, 0:1] + d[:, 1:2] + 1.0
    dinv = lax.rsqrt(degsum)
    feat_ref[...] = feat
    xs1_ref[...] = feat * dinv
    dinv_ref[...] = dinv


def _tc_encoder(x_pad, deg_t, w0, b0, w1, b1):
    grid = (N_PAD // BLK,)
    return pl.pallas_call(
        _enc_body,
        grid=grid,
        in_specs=[
            pl.BlockSpec((BLK, 128), lambda i: (i, 0)),
            pl.BlockSpec((BLK, NC), lambda i: (i, 0)),
            pl.BlockSpec((128, 32), lambda i: (0, 0)),
            pl.BlockSpec((1, 32), lambda i: (0, 0)),
            pl.BlockSpec((32, 32), lambda i: (0, 0)),
            pl.BlockSpec((1, 32), lambda i: (0, 0)),
        ],
        out_specs=[
            pl.BlockSpec((BLK, 32), lambda i: (i, 0)),
            pl.BlockSpec((BLK, 32), lambda i: (i, 0)),
            pl.BlockSpec((BLK, 1), lambda i: (i, 0)),
        ],
        out_shape=[
            jax.ShapeDtypeStruct((N_PAD, 32), jnp.float32),
            jax.ShapeDtypeStruct((N_PAD, 32), jnp.float32),
            jax.ShapeDtypeStruct((N_PAD, 1), jnp.float32),
        ],
    )(x_pad, deg_t, w0, b0, w1, b1)


def _mid_body(p1_ref, xs1_ref, dinv_ref, wc_ref, bc_ref, wcat_ref, xs2_ref):
    a = p1_ref[...]
    pf = (a[0] + a[1] + xs1_ref[...]) * dinv_ref[...]
    cc = jnp.maximum(
        jnp.dot(pf, wc_ref[...], preferred_element_type=jnp.float32)
        + bc_ref[...], 0.0)
    t = jnp.dot(cc, wcat_ref[...], preferred_element_type=jnp.float32)
    xs2_ref[...] = t * dinv_ref[...]


def _tc_mid(p1, xs1, dinv, wc, bc, wcat):
    grid = (N_PAD // BLK,)
    return pl.pallas_call(
        _mid_body,
        grid=grid,
        in_specs=[
            pl.BlockSpec((NC, BLK, 32), lambda i: (0, i, 0)),
            pl.BlockSpec((BLK, 32), lambda i: (i, 0)),
            pl.BlockSpec((BLK, 1), lambda i: (i, 0)),
            pl.BlockSpec((32, 64), lambda i: (0, 0)),
            pl.BlockSpec((1, 64), lambda i: (0, 0)),
            pl.BlockSpec((64, 16), lambda i: (0, 0)),
        ],
        out_specs=pl.BlockSpec((BLK, 16), lambda i: (i, 0)),
        out_shape=jax.ShapeDtypeStruct((N_PAD, 16), jnp.float32),
    )(p1, xs1, dinv, wc, bc, wcat)


def _tail_body(p2_ref, xs2_ref, dinv_ref, feat_ref, bcat_ref,
               wd0a_ref, wd0b_ref, bd0_ref, wd1_ref, bd1_ref,
               cla_ref, clb_ref, csq_ref,
               mu_ref, lv_ref, de_ref, q_ref):
    a = p2_ref[...]
    pt = (a[0] + a[1] + xs2_ref[...]) * dinv_ref[...] + bcat_ref[...]
    mu = pt[:, 0:8]
    feat = feat_ref[...]
    dh = _elu(
        jnp.dot(feat, wd0a_ref[...], preferred_element_type=jnp.float32)
        + jnp.dot(mu, wd0b_ref[...], preferred_element_type=jnp.float32)
        + bd0_ref[...])
    de = jnp.dot(dh, wd1_ref[...],
                 preferred_element_type=jnp.float32) + bd1_ref[...]
    zsq = (jnp.sum(feat * feat, axis=1, keepdims=True)
           + jnp.sum(mu * mu, axis=1, keepdims=True))
    zc = (jnp.dot(feat, cla_ref[...], preferred_element_type=jnp.float32)
          + jnp.dot(mu, clb_ref[...], preferred_element_type=jnp.float32))
    qd = zsq - 2.0 * zc + csq_ref[...]
    u = 1.0 / (1.0 + qd * (1.0 / ALPHA))
    qq = jnp.exp(QEXP * jnp.log(u))
    mu_ref[...] = mu
    lv_ref[...] = pt[:, 8:16]
    de_ref[...] = de
    q_ref[...] = qq / jnp.sum(qq, axis=1, keepdims=True)


def _tc_tail(p2, xs2, dinv, feat, bcat, wd0a, wd0b, bd0, wd1, bd1,
             cla, clb, csq):
    grid = (N_PAD // BLK,)
    return pl.pallas_call(
        _tail_body,
        grid=grid,
        in_specs=[
            pl.BlockSpec((NC, BLK, 16), lambda i: (0, i, 0)),
            pl.BlockSpec((BLK, 16), lambda i: (i, 0)),
            pl.BlockSpec((BLK, 1), lambda i: (i, 0)),
            pl.BlockSpec((BLK, 32), lambda i: (i, 0)),
            pl.BlockSpec((1, 16), lambda i: (0, 0)),
            pl.BlockSpec((32, 32), lambda i: (0, 0)),
            pl.BlockSpec((8, 32), lambda i: (0, 0)),
            pl.BlockSpec((1, 32), lambda i: (0, 0)),
            pl.BlockSpec((32, 128), lambda i: (0, 0)),
            pl.BlockSpec((1, 128), lambda i: (0, 0)),
            pl.BlockSpec((32, 15), lambda i: (0, 0)),
            pl.BlockSpec((8, 15), lambda i: (0, 0)),
            pl.BlockSpec((1, 15), lambda i: (0, 0)),
        ],
        out_specs=[
            pl.BlockSpec((BLK, 8), lambda i: (i, 0)),
            pl.BlockSpec((BLK, 8), lambda i: (i, 0)),
            pl.BlockSpec((BLK, 128), lambda i: (i, 0)),
            pl.BlockSpec((BLK, 15), lambda i: (i, 0)),
        ],
        out_shape=[
            jax.ShapeDtypeStruct((N_PAD, 8), jnp.float32),
            jax.ShapeDtypeStruct((N_PAD, 8), jnp.float32),
            jax.ShapeDtypeStruct((N_PAD, 128), jnp.float32),
            jax.ShapeDtypeStruct((N_PAD, 15), jnp.float32),
        ],
    )(p2, xs2, dinv, feat, bcat, wd0a, wd0b, bd0, wd1, bd1, cla, clb, csq)


def kernel(x, params, adj):
    p = params
    s_bn = (1.0 + EPS_BN) ** -0.5

    w0 = p["enc_W0"] * (p["enc_g0"] * s_bn)[None, :]
    b0 = (p["enc_b0"] * p["enc_g0"] * s_bn + p["enc_be0"])[None, :]
    w1 = p["enc_W1"] * (p["enc_g1"] * s_bn)[None, :]
    b1 = (p["enc_b1"] * p["enc_g1"] * s_bn + p["enc_be1"])[None, :]
    w1p = jnp.pad(w1, ((0, 0), (0, 12)))
    b1p = jnp.pad(b1, ((0, 0), (0, 12)))
    wc = p["conv_W"] * (p["conv_g"] * s_bn)[None, :]
    bc = (p["conv_b"] * p["conv_g"] * s_bn + p["conv_be"])[None, :]
    wcp = jnp.pad(wc, ((0, 12), (0, 0)))
    wcat = jnp.concatenate([p["mean_W"], p["logv_W"]], axis=1)
    bcat = jnp.concatenate([p["mean_b"], p["logv_b"]])[None, :]
    wd0 = p["dec_W0"] * (p["dec_g0"] * s_bn)[None, :]
    bd0 = (p["dec_b0"] * p["dec_g0"] * s_bn + p["dec_be0"])[None, :]
    wd0a = jnp.pad(wd0[:20], ((0, 12), (0, 0)))
    wd0b = wd0[20:]
    wd1 = p["dec_W1"]
    bd1 = p["dec_b1"][None, :]
    cl = p["cluster"]
    cla = jnp.pad(cl[:, :20].T, ((0, 12), (0, 0)))
    clb = cl[:, 20:].T
    csq = jnp.sum(cl * cl, axis=1)[None, :]

    src = adj[0]
    dst = adj[1]
    ar = jnp.arange(E_PAD - E, dtype=jnp.int32)
    src_pad = jnp.concatenate([src, ar % 997]).reshape(NW, CPW, CHUNK)
    dst_pad = jnp.concatenate([dst, N + ar % (N_PAD - N)]).reshape(NW, CPW, CHUNK)

    x_pad = jnp.pad(x, ((0, N_PAD - N), (0, 0)))

    deg = _sc_degree(dst_pad)
    feat, xs1, dinv = _tc_encoder(x_pad, deg.T, w0, b0, w1p, b1p)
    p1 = _sc_propagate(xs1, src_pad, dst_pad, 32)
    xs2 = _tc_mid(p1, xs1, dinv, wcp, bc, wcat)
    p2 = _sc_propagate(xs2, src_pad, dst_pad, 16)
    mu, logvar, de_feat, q = _tc_tail(
        p2, xs2, dinv, feat, bcat, wd0a, wd0b, bd0, wd1, bd1, cla, clb, csq)

    mu_n = mu[:N]
    feat_n = feat[:N, :20]
    z = jnp.concatenate([feat_n, mu_n], axis=1)
    return (z, mu_n, logvar[:N], de_feat[:N], q[:N], feat_n, mu_n)


# speedup vs baseline: 60.2815x; 1.0122x over previous
import functools

import jax
import jax.numpy as jnp
from jax import lax
from jax.experimental import pallas as pl
from jax.experimental.pallas import tpu as pltpu
from jax.experimental.pallas import tpu_sc as plsc

N = 10000
E = 320000
N_PAD = 10240
NC = 2
NS = 16
NW = NC * NS
CHUNK = 128
CPW = 80
E_PAD = NW * CPW * CHUNK
ACC_ROWS_PER_TILE = N_PAD // NS
OUT_ROWS_PER_TILE = N // NS

ALPHA = 0.9
EPS_BN = 0.001
QEXP = (ALPHA + 1.0) / 2.0

BLK = 2000


def _sc_degree(dst_idx):
    mesh = plsc.VectorSubcoreMesh(
        core_axis_name="c", subcore_axis_name="s", num_cores=NC, num_subcores=NS)

    @functools.partial(
        pl.kernel,
        out_type=jax.ShapeDtypeStruct((NC, N_PAD), jnp.float32),
        mesh=mesh,
        compiler_params=pltpu.CompilerParams(use_tc_tiling_on_sc=False),
        scratch_types=[
            pltpu.VMEM((CPW, CHUNK), jnp.int32),
            pltpu.VMEM((CHUNK,), jnp.float32),
            pltpu.VMEM_SHARED((N_PAD,), jnp.float32),
            pltpu.SemaphoreType.DMA,
        ],
    )
    def k(dst_hbm, out_hbm, idx_v, buf_v, acc, sem):
        c = lax.axis_index("c")
        s = lax.axis_index("s")
        w = s * NC + c
        pltpu.sync_copy(dst_hbm.at[w], idx_v)
        for i in range(CHUNK // 16):
            buf_v[pl.ds(i * 16, 16)] = jnp.zeros((16,), jnp.float32)
        for kk in range(ACC_ROWS_PER_TILE // CHUNK):
            pltpu.sync_copy(
                buf_v, acc.at[pl.ds(s * ACC_ROWS_PER_TILE + kk * CHUNK, CHUNK)])
        for i in range(CHUNK // 16):
            buf_v[pl.ds(i * 16, 16)] = jnp.ones((16,), jnp.float32)
        plsc.subcore_barrier()

        grp = 16

        def body(g, carry):
            for i in range(grp):
                pltpu.async_copy(buf_v, acc.at[idx_v.at[g * grp + i]], sem,
                                 add=True)
            for i in range(grp):
                pltpu.make_async_copy(
                    buf_v, acc.at[idx_v.at[g * grp + i]], sem).wait()
            return carry

        lax.fori_loop(0, CPW // grp, body, 0)
        plsc.subcore_barrier()
        pltpu.sync_copy(
            acc.at[pl.ds(s * ACC_ROWS_PER_TILE, ACC_ROWS_PER_TILE)],
            out_hbm.at[c, pl.ds(s * ACC_ROWS_PER_TILE, ACC_ROWS_PER_TILE)])

    return k(dst_idx)


def _sc_propagate(xs, src_idx, dst_idx, f):
    mesh = plsc.VectorSubcoreMesh(
        core_axis_name="c", subcore_axis_name="s", num_cores=NC, num_subcores=NS)

    @functools.partial(
        pl.kernel,
        out_type=jax.ShapeDtypeStruct((NC, N, f), jnp.float32),
        mesh=mesh,
        compiler_params=pltpu.CompilerParams(use_tc_tiling_on_sc=False),
        scratch_types=[
            pltpu.VMEM((CPW, CHUNK), jnp.int32),
            pltpu.VMEM((CPW, CHUNK), jnp.int32),
            pltpu.VMEM((4, CHUNK, f), jnp.float32),
            pltpu.VMEM_SHARED((N_PAD, f), jnp.float32),
            [pltpu.SemaphoreType.DMA] * 4,
            [pltpu.SemaphoreType.DMA] * 4,
        ],
    )
    def k(xs_hbm, src_hbm, dst_hbm, out_hbm, src_v, dst_v, rows_v, acc,
          gsem, ssem):
        c = lax.axis_index("c")
        s = lax.axis_index("s")
        w = s * NC + c
        pltpu.sync_copy(src_hbm.at[w], src_v)
        pltpu.sync_copy(dst_hbm.at[w], dst_v)

        zero16 = jnp.zeros((16,), jnp.float32)

        def zrow(r, carry):
            for i in range(f // 16):
                rows_v[0, r, pl.ds(i * 16, 16)] = zero16
            return carry

        lax.fori_loop(0, CHUNK, zrow, 0)
        for kk in range(ACC_ROWS_PER_TILE // CHUNK):
            pltpu.sync_copy(
                rows_v.at[0],
                acc.at[pl.ds(s * ACC_ROWS_PER_TILE + kk * CHUNK, CHUNK)])
        plsc.subcore_barrier()

        for b in range(4):
            pltpu.async_copy(xs_hbm.at[src_v.at[b]], rows_v.at[b], gsem[b])

        def body(jo, carry):
            j0 = 4 * jo
            for b in range(4):
                j = j0 + b
                pltpu.make_async_copy(
                    xs_hbm.at[src_v.at[j]], rows_v.at[b], gsem[b]).wait()
                pltpu.async_copy(rows_v.at[b], acc.at[dst_v.at[j]], ssem[b],
                                 add=True)
            for b in range(4):
                j = j0 + b

                @pl.when(j + 4 < CPW)
                def _():
                    pltpu.make_async_copy(
                        rows_v.at[b], acc.at[dst_v.at[j]], ssem[b]).wait()
                    pltpu.async_copy(
                        xs_hbm.at[src_v.at[j + 4]], rows_v.at[b], gsem[b])
            return carry

        lax.fori_loop(0, CPW // 4, body, 0)
        for b in range(4):
            j = CPW - 4 + b
            pltpu.make_async_copy(
                rows_v.at[b], acc.at[dst_v.at[j]], ssem[b]).wait()

        plsc.subcore_barrier()
        pltpu.sync_copy(
            acc.at[pl.ds(s * OUT_ROWS_PER_TILE, OUT_ROWS_PER_TILE)],
            out_hbm.at[c].at[pl.ds(s * OUT_ROWS_PER_TILE, OUT_ROWS_PER_TILE)])

    return k(xs, src_idx, dst_idx)


def _elu(v):
    return jnp.where(v > 0.0, v, jnp.exp(v) - 1.0)


def _enc_body(x_ref, w0_ref, b0_ref, w1_ref, b1_ref, feat_ref):
    h = _elu(jnp.dot(x_ref[...], w0_ref[...],
                     preferred_element_type=jnp.float32) + b0_ref[...])
    feat_ref[...] = _elu(
        jnp.dot(h, w1_ref[...], preferred_element_type=jnp.float32)
        + b1_ref[...])


def _tc_encoder(x, w0, b0, w1, b1):
    return pl.pallas_call(
        _enc_body,
        grid=(N // BLK,),
        in_specs=[
            pl.BlockSpec((BLK, 128), lambda i: (i, 0)),
            pl.BlockSpec((128, 32), lambda i: (0, 0)),
            pl.BlockSpec((1, 32), lambda i: (0, 0)),
            pl.BlockSpec((32, 32), lambda i: (0, 0)),
            pl.BlockSpec((1, 32), lambda i: (0, 0)),
        ],
        out_specs=pl.BlockSpec((BLK, 32), lambda i: (i, 0)),
        out_shape=jax.ShapeDtypeStruct((N, 32), jnp.float32),
    )(x, w0, b0, w1, b1)


def _scale_body(feat_ref, degt_ref, xs1_ref, dinv_ref):
    d = degt_ref[...]
    dinv = lax.rsqrt(d[:, 0:1] + d[:, 1:2] + 1.0)
    xs1_ref[...] = feat_ref[...] * dinv
    dinv_ref[...] = dinv


def _tc_scale(feat, deg_t):
    return pl.pallas_call(
        _scale_body,
        grid=(N // BLK,),
        in_specs=[
            pl.BlockSpec((BLK, 32), lambda i: (i, 0)),
            pl.BlockSpec((BLK, NC), lambda i: (i, 0)),
        ],
        out_specs=[
            pl.BlockSpec((BLK, 32), lambda i: (i, 0)),
            pl.BlockSpec((BLK, 1), lambda i: (i, 0)),
        ],
        out_shape=[
            jax.ShapeDtypeStruct((N, 32), jnp.float32),
            jax.ShapeDtypeStruct((N, 1), jnp.float32),
        ],
    )(feat, deg_t)


def _mid_body(p1_ref, xs1_ref, dinv_ref, wc_ref, bc_ref, wcat_ref, xs2_ref):
    a = p1_ref[...]
    pf = (a[0] + a[1] + xs1_ref[...]) * dinv_ref[...]
    cc = jnp.maximum(
        jnp.dot(pf, wc_ref[...], preferred_element_type=jnp.float32)
        + bc_ref[...], 0.0)
    t = jnp.dot(cc, wcat_ref[...], preferred_element_type=jnp.float32)
    xs2_ref[...] = t * dinv_ref[...]


def _tc_mid(p1, xs1, dinv, wc, bc, wcat):
    return pl.pallas_call(
        _mid_body,
        grid=(N // BLK,),
        in_specs=[
            pl.BlockSpec((NC, BLK, 32), lambda i: (0, i, 0)),
            pl.BlockSpec((BLK, 32), lambda i: (i, 0)),
            pl.BlockSpec((BLK, 1), lambda i: (i, 0)),
            pl.BlockSpec((32, 64), lambda i: (0, 0)),
            pl.BlockSpec((1, 64), lambda i: (0, 0)),
            pl.BlockSpec((64, 16), lambda i: (0, 0)),
        ],
        out_specs=pl.BlockSpec((BLK, 16), lambda i: (i, 0)),
        out_shape=jax.ShapeDtypeStruct((N, 16), jnp.float32),
    )(p1, xs1, dinv, wc, bc, wcat)


def _tail_body(p2_ref, xs2_ref, dinv_ref, feat_ref, bcat_ref,
               wd0a_ref, wd0b_ref, bd0_ref, wd1_ref, bd1_ref,
               cla_ref, clb_ref, csq_ref,
               z_ref, mu_ref, lv_ref, de_ref, q_ref, f20_ref):
    a = p2_ref[...]
    pt = (a[0] + a[1] + xs2_ref[...]) * dinv_ref[...] + bcat_ref[...]
    mu = pt[:, 0:8]
    feat = feat_ref[...]
    dh = _elu(
        jnp.dot(feat, wd0a_ref[...], preferred_element_type=jnp.float32)
        + jnp.dot(mu, wd0b_ref[...], preferred_element_type=jnp.float32)
        + bd0_ref[...])
    de = jnp.dot(dh, wd1_ref[...],
                 preferred_element_type=jnp.float32) + bd1_ref[...]
    zsq = (jnp.sum(feat * feat, axis=1, keepdims=True)
           + jnp.sum(mu * mu, axis=1, keepdims=True))
    zc = (jnp.dot(feat, cla_ref[...], preferred_element_type=jnp.float32)
          + jnp.dot(mu, clb_ref[...], preferred_element_type=jnp.float32))
    qd = zsq - 2.0 * zc + csq_ref[...]
    u = 1.0 / (1.0 + qd * (1.0 / ALPHA))
    qq = jnp.exp(QEXP * jnp.log(u))
    f20 = feat[:, 0:20]
    z_ref[...] = jnp.concatenate([f20, mu], axis=1)
    mu_ref[...] = mu
    lv_ref[...] = pt[:, 8:16]
    de_ref[...] = de
    q_ref[...] = qq / jnp.sum(qq, axis=1, keepdims=True)
    f20_ref[...] = f20


def _tc_tail(p2, xs2, dinv, feat, bcat, wd0a, wd0b, bd0, wd1, bd1,
             cla, clb, csq):
    return pl.pallas_call(
        _tail_body,
        grid=(N // BLK,),
        in_specs=[
            pl.BlockSpec((NC, BLK, 16), lambda i: (0, i, 0)),
            pl.BlockSpec((BLK, 16), lambda i: (i, 0)),
            pl.BlockSpec((BLK, 1), lambda i: (i, 0)),
            pl.BlockSpec((BLK, 32), lambda i: (i, 0)),
            pl.BlockSpec((1, 16), lambda i: (0, 0)),
            pl.BlockSpec((32, 32), lambda i: (0, 0)),
            pl.BlockSpec((8, 32), lambda i: (0, 0)),
            pl.BlockSpec((1, 32), lambda i: (0, 0)),
            pl.BlockSpec((32, 128), lambda i: (0, 0)),
            pl.BlockSpec((1, 128), lambda i: (0, 0)),
            pl.BlockSpec((32, 15), lambda i: (0, 0)),
            pl.BlockSpec((8, 15), lambda i: (0, 0)),
            pl.BlockSpec((1, 15), lambda i: (0, 0)),
        ],
        out_specs=[
            pl.BlockSpec((BLK, 28), lambda i: (i, 0)),
            pl.BlockSpec((BLK, 8), lambda i: (i, 0)),
            pl.BlockSpec((BLK, 8), lambda i: (i, 0)),
            pl.BlockSpec((BLK, 128), lambda i: (i, 0)),
            pl.BlockSpec((BLK, 15), lambda i: (i, 0)),
            pl.BlockSpec((BLK, 20), lambda i: (i, 0)),
        ],
        out_shape=[
            jax.ShapeDtypeStruct((N, 28), jnp.float32),
            jax.ShapeDtypeStruct((N, 8), jnp.float32),
            jax.ShapeDtypeStruct((N, 8), jnp.float32),
            jax.ShapeDtypeStruct((N, 128), jnp.float32),
            jax.ShapeDtypeStruct((N, 15), jnp.float32),
            jax.ShapeDtypeStruct((N, 20), jnp.float32),
        ],
    )(p2, xs2, dinv, feat, bcat, wd0a, wd0b, bd0, wd1, bd1, cla, clb, csq)


def kernel(x, params, adj):
    p = params
    s_bn = (1.0 + EPS_BN) ** -0.5

    w0 = p["enc_W0"] * (p["enc_g0"] * s_bn)[None, :]
    b0 = (p["enc_b0"] * p["enc_g0"] * s_bn + p["enc_be0"])[None, :]
    w1 = p["enc_W1"] * (p["enc_g1"] * s_bn)[None, :]
    b1 = (p["enc_b1"] * p["enc_g1"] * s_bn + p["enc_be1"])[None, :]
    w1p = jnp.pad(w1, ((0, 0), (0, 12)))
    b1p = jnp.pad(b1, ((0, 0), (0, 12)))
    wc = p["conv_W"] * (p["conv_g"] * s_bn)[None, :]
    bc = (p["conv_b"] * p["conv_g"] * s_bn + p["conv_be"])[None, :]
    wcp = jnp.pad(wc, ((0, 12), (0, 0)))
    wcat = jnp.concatenate([p["mean_W"], p["logv_W"]], axis=1)
    bcat = jnp.concatenate([p["mean_b"], p["logv_b"]])[None, :]
    wd0 = p["dec_W0"] * (p["dec_g0"] * s_bn)[None, :]
    bd0 = (p["dec_b0"] * p["dec_g0"] * s_bn + p["dec_be0"])[None, :]
    wd0a = jnp.pad(wd0[:20], ((0, 12), (0, 0)))
    wd0b = wd0[20:]
    wd1 = p["dec_W1"]
    bd1 = p["dec_b1"][None, :]
    cl = p["cluster"]
    cla = jnp.pad(cl[:, :20].T, ((0, 12), (0, 0)))
    clb = cl[:, 20:].T
    csq = jnp.sum(cl * cl, axis=1)[None, :]

    src = adj[0]
    dst = adj[1]
    ar = jnp.arange(E_PAD - E, dtype=jnp.int32)
    src_pad = jnp.concatenate([src, ar % 997]).reshape(NW, CPW, CHUNK)
    dst_pad = jnp.concatenate([dst, N + ar % (N_PAD - N)]).reshape(NW, CPW, CHUNK)

    deg = _sc_degree(dst_pad)
    feat = _tc_encoder(x, w0, b0, w1p, b1p)
    xs1, dinv = _tc_scale(feat, deg.T[:N])
    p1 = _sc_propagate(xs1, src_pad, dst_pad, 32)
    xs2 = _tc_mid(p1, xs1, dinv, wcp, bc, wcat)
    p2 = _sc_propagate(xs2, src_pad, dst_pad, 16)
    z, mu, logvar, de_feat, q, feat20 = _tc_tail(
        p2, xs2, dinv, feat, bcat, wd0a, wd0b, bd0, wd1, bd1, cla, clb, csq)

    return (z, mu, logvar, de_feat, q, feat20, mu)


# speedup vs baseline: 68.7499x; 1.1405x over previous
import functools

import jax
import jax.numpy as jnp
from jax import lax
from jax.experimental import pallas as pl
from jax.experimental.pallas import tpu as pltpu
from jax.experimental.pallas import tpu_sc as plsc

N = 10000
E = 320000
N_PAD = 10240
NC = 2
NS = 16
NW = NC * NS
CHUNK = 128
CPW = 80
E_PAD = NW * CPW * CHUNK
ACC_ROWS_PER_TILE = N_PAD // NS
OUT_ROWS_PER_TILE = N // NS

ALPHA = 0.9
EPS_BN = 0.001
QEXP = (ALPHA + 1.0) / 2.0

BLK = 2000


def _sc_degree(dst_idx):
    mesh = plsc.VectorSubcoreMesh(
        core_axis_name="c", subcore_axis_name="s", num_cores=NC, num_subcores=NS)

    @functools.partial(
        pl.kernel,
        out_type=jax.ShapeDtypeStruct((NC, N_PAD), jnp.float32),
        mesh=mesh,
        compiler_params=pltpu.CompilerParams(use_tc_tiling_on_sc=False),
        scratch_types=[
            pltpu.VMEM((CPW, CHUNK), jnp.int32),
            pltpu.VMEM((CHUNK,), jnp.float32),
            pltpu.VMEM_SHARED((N_PAD,), jnp.float32),
            pltpu.SemaphoreType.DMA,
        ],
    )
    def k(dst_hbm, out_hbm, idx_v, buf_v, acc, sem):
        c = lax.axis_index("c")
        s = lax.axis_index("s")
        w = s * NC + c
        pltpu.sync_copy(dst_hbm.at[w], idx_v)
        for i in range(CHUNK // 16):
            buf_v[pl.ds(i * 16, 16)] = jnp.zeros((16,), jnp.float32)
        for kk in range(ACC_ROWS_PER_TILE // CHUNK):
            pltpu.sync_copy(
                buf_v, acc.at[pl.ds(s * ACC_ROWS_PER_TILE + kk * CHUNK, CHUNK)])
        for i in range(CHUNK // 16):
            buf_v[pl.ds(i * 16, 16)] = jnp.ones((16,), jnp.float32)
        plsc.subcore_barrier()

        grp = 16

        def body(g, carry):
            for i in range(grp):
                pltpu.async_copy(buf_v, acc.at[idx_v.at[g * grp + i]], sem,
                                 add=True)
            for i in range(grp):
                pltpu.make_async_copy(
                    buf_v, acc.at[idx_v.at[g * grp + i]], sem).wait()
            return carry

        lax.fori_loop(0, CPW // grp, body, 0)
        plsc.subcore_barrier()
        pltpu.sync_copy(
            acc.at[pl.ds(s * ACC_ROWS_PER_TILE, ACC_ROWS_PER_TILE)],
            out_hbm.at[c, pl.ds(s * ACC_ROWS_PER_TILE, ACC_ROWS_PER_TILE)])

    return k(dst_idx)


def _sc_propagate(xs, src_idx, dst_idx, f):
    mesh = plsc.VectorSubcoreMesh(
        core_axis_name="c", subcore_axis_name="s", num_cores=NC, num_subcores=NS)

    @functools.partial(
        pl.kernel,
        out_type=jax.ShapeDtypeStruct((NC, N, f), jnp.float32),
        mesh=mesh,
        compiler_params=pltpu.CompilerParams(use_tc_tiling_on_sc=False),
        scratch_types=[
            pltpu.VMEM((CPW, CHUNK), jnp.int32),
            pltpu.VMEM((CPW, CHUNK), jnp.int32),
            pltpu.VMEM((4, CHUNK, f), jnp.float32),
            pltpu.VMEM_SHARED((N_PAD, f), jnp.float32),
            [pltpu.SemaphoreType.DMA] * 4,
            [pltpu.SemaphoreType.DMA] * 4,
        ],
    )
    def k(xs_hbm, src_hbm, dst_hbm, out_hbm, src_v, dst_v, rows_v, acc,
          gsem, ssem):
        c = lax.axis_index("c")
        s = lax.axis_index("s")
        w = s * NC + c
        pltpu.sync_copy(src_hbm.at[w], src_v)
        pltpu.sync_copy(dst_hbm.at[w], dst_v)

        zero16 = jnp.zeros((16,), jnp.float32)

        def zrow(r, carry):
            for i in range(f // 16):
                rows_v[0, r, pl.ds(i * 16, 16)] = zero16
            return carry

        lax.fori_loop(0, CHUNK, zrow, 0)
        for kk in range(ACC_ROWS_PER_TILE // CHUNK):
            pltpu.sync_copy(
                rows_v.at[0],
                acc.at[pl.ds(s * ACC_ROWS_PER_TILE + kk * CHUNK, CHUNK)])
        plsc.subcore_barrier()

        for b in range(4):
            pltpu.async_copy(xs_hbm.at[src_v.at[b]], rows_v.at[b], gsem[b])

        def body(jo, carry):
            j0 = 4 * jo
            for b in range(4):
                j = j0 + b
                pltpu.make_async_copy(
                    xs_hbm.at[src_v.at[j]], rows_v.at[b], gsem[b]).wait()
                pltpu.async_copy(rows_v.at[b], acc.at[dst_v.at[j]], ssem[b],
                                 add=True)
            for b in range(4):
                j = j0 + b

                @pl.when(j + 4 < CPW)
                def _():
                    pltpu.make_async_copy(
                        rows_v.at[b], acc.at[dst_v.at[j]], ssem[b]).wait()
                    pltpu.async_copy(
                        xs_hbm.at[src_v.at[j + 4]], rows_v.at[b], gsem[b])
            return carry

        lax.fori_loop(0, CPW // 4, body, 0)
        for b in range(4):
            j = CPW - 4 + b
            pltpu.make_async_copy(
                rows_v.at[b], acc.at[dst_v.at[j]], ssem[b]).wait()

        plsc.subcore_barrier()
        pltpu.sync_copy(
            acc.at[pl.ds(s * OUT_ROWS_PER_TILE, OUT_ROWS_PER_TILE)],
            out_hbm.at[c].at[pl.ds(s * OUT_ROWS_PER_TILE, OUT_ROWS_PER_TILE)])

    return k(xs, src_idx, dst_idx)


def _elu(v):
    return jnp.where(v > 0.0, v, jnp.exp(v) - 1.0)


def _enc_body(x_ref, degt_ref, w0_ref, b0_ref, w1_ref, b1_ref,
              feat_ref, xs1_ref, dinv_ref):
    h = _elu(jnp.dot(x_ref[...], w0_ref[...],
                     preferred_element_type=jnp.float32) + b0_ref[...])
    feat = _elu(
        jnp.dot(h, w1_ref[...], preferred_element_type=jnp.float32)
        + b1_ref[...])
    d = degt_ref[...]
    dinv = lax.rsqrt(d[:, 0:1] + d[:, 1:2] + 1.0)
    feat_ref[...] = feat
    xs1_ref[...] = feat * dinv
    dinv_ref[...] = dinv


def _tc_encoder(x, deg_t, w0, b0, w1, b1):
    return pl.pallas_call(
        _enc_body,
        grid=(N // BLK,),
        in_specs=[
            pl.BlockSpec((BLK, 128), lambda i: (i, 0)),
            pl.BlockSpec((BLK, NC), lambda i: (i, 0)),
            pl.BlockSpec((128, 32), lambda i: (0, 0)),
            pl.BlockSpec((1, 32), lambda i: (0, 0)),
            pl.BlockSpec((32, 32), lambda i: (0, 0)),
            pl.BlockSpec((1, 32), lambda i: (0, 0)),
        ],
        out_specs=[
            pl.BlockSpec((BLK, 32), lambda i: (i, 0)),
            pl.BlockSpec((BLK, 32), lambda i: (i, 0)),
            pl.BlockSpec((BLK, 1), lambda i: (i, 0)),
        ],
        out_shape=[
            jax.ShapeDtypeStruct((N, 32), jnp.float32),
            jax.ShapeDtypeStruct((N, 32), jnp.float32),
            jax.ShapeDtypeStruct((N, 1), jnp.float32),
        ],
    )(x, deg_t, w0, b0, w1, b1)


def _mid_body(p1_ref, xs1_ref, dinv_ref, wc_ref, bc_ref, wcat_ref, xs2_ref):
    a = p1_ref[...]
    pf = (a[0] + a[1] + xs1_ref[...]) * dinv_ref[...]
    cc = jnp.maximum(
        jnp.dot(pf, wc_ref[...], preferred_element_type=jnp.float32)
        + bc_ref[...], 0.0)
    t = jnp.dot(cc, wcat_ref[...], preferred_element_type=jnp.float32)
    xs2_ref[...] = t * dinv_ref[...]


def _tc_mid(p1, xs1, dinv, wc, bc, wcat):
    return pl.pallas_call(
        _mid_body,
        grid=(N // BLK,),
        in_specs=[
            pl.BlockSpec((NC, BLK, 32), lambda i: (0, i, 0)),
            pl.BlockSpec((BLK, 32), lambda i: (i, 0)),
            pl.BlockSpec((BLK, 1), lambda i: (i, 0)),
            pl.BlockSpec((32, 64), lambda i: (0, 0)),
            pl.BlockSpec((1, 64), lambda i: (0, 0)),
            pl.BlockSpec((64, 16), lambda i: (0, 0)),
        ],
        out_specs=pl.BlockSpec((BLK, 16), lambda i: (i, 0)),
        out_shape=jax.ShapeDtypeStruct((N, 16), jnp.float32),
    )(p1, xs1, dinv, wc, bc, wcat)


def _tail_body(p2_ref, xs2_ref, dinv_ref, feat_ref, bcatt_ref,
               wd0t_ref, bd0t_ref, wd1_ref, bd1_ref, cl_ref, csqt_ref,
               zt_ref, mut_ref, lvt_ref, de_ref, qt_ref, f20t_ref):
    a = p2_ref[...]
    pt = (a[0] + a[1] + xs2_ref[...]) * dinv_ref[...]
    ptt = jnp.transpose(pt) + bcatt_ref[...]
    mut = ptt[0:8, :]
    featt = jnp.transpose(feat_ref[...])
    f20t = featt[0:20, :]
    zt = jnp.concatenate([f20t, mut], axis=0)
    dht = _elu(
        jnp.dot(wd0t_ref[...], zt, preferred_element_type=jnp.float32)
        + bd0t_ref[...])
    de = jnp.dot(jnp.transpose(dht), wd1_ref[...],
                 preferred_element_type=jnp.float32) + bd1_ref[...]
    zsqt = jnp.sum(zt * zt, axis=0, keepdims=True)
    zct = jnp.dot(cl_ref[...], zt, preferred_element_type=jnp.float32)
    qdt = zsqt - 2.0 * zct + csqt_ref[...]
    u = 1.0 / (1.0 + qdt * (1.0 / ALPHA))
    qq = jnp.exp(QEXP * jnp.log(u))
    zt_ref[...] = zt
    mut_ref[...] = mut
    lvt_ref[...] = ptt[8:16, :]
    de_ref[...] = de
    qt_ref[...] = qq / jnp.sum(qq, axis=0, keepdims=True)
    f20t_ref[...] = f20t


def _tc_tail(p2, xs2, dinv, feat, bcat_t, wd0_t, bd0_t, wd1, bd1, cl, csq_t):
    return pl.pallas_call(
        _tail_body,
        in_specs=[
            pl.BlockSpec((NC, N, 16), lambda: (0, 0, 0)),
            pl.BlockSpec((N, 16), lambda: (0, 0)),
            pl.BlockSpec((N, 1), lambda: (0, 0)),
            pl.BlockSpec((N, 32), lambda: (0, 0)),
            pl.BlockSpec((16, 1), lambda: (0, 0)),
            pl.BlockSpec((32, 28), lambda: (0, 0)),
            pl.BlockSpec((32, 1), lambda: (0, 0)),
            pl.BlockSpec((32, 128), lambda: (0, 0)),
            pl.BlockSpec((1, 128), lambda: (0, 0)),
            pl.BlockSpec((15, 28), lambda: (0, 0)),
            pl.BlockSpec((15, 1), lambda: (0, 0)),
        ],
        out_specs=[
            pl.BlockSpec((28, N), lambda: (0, 0)),
            pl.BlockSpec((8, N), lambda: (0, 0)),
            pl.BlockSpec((8, N), lambda: (0, 0)),
            pl.BlockSpec((N, 128), lambda: (0, 0)),
            pl.BlockSpec((15, N), lambda: (0, 0)),
            pl.BlockSpec((20, N), lambda: (0, 0)),
        ],
        out_shape=[
            jax.ShapeDtypeStruct((28, N), jnp.float32),
            jax.ShapeDtypeStruct((8, N), jnp.float32),
            jax.ShapeDtypeStruct((8, N), jnp.float32),
            jax.ShapeDtypeStruct((N, 128), jnp.float32),
            jax.ShapeDtypeStruct((15, N), jnp.float32),
            jax.ShapeDtypeStruct((20, N), jnp.float32),
        ],
    )(p2, xs2, dinv, feat, bcat_t, wd0_t, bd0_t, wd1, bd1, cl, csq_t)


def kernel(x, params, adj):
    p = params
    s_bn = (1.0 + EPS_BN) ** -0.5

    w0 = p["enc_W0"] * (p["enc_g0"] * s_bn)[None, :]
    b0 = (p["enc_b0"] * p["enc_g0"] * s_bn + p["enc_be0"])[None, :]
    w1 = p["enc_W1"] * (p["enc_g1"] * s_bn)[None, :]
    b1 = (p["enc_b1"] * p["enc_g1"] * s_bn + p["enc_be1"])[None, :]
    w1p = jnp.pad(w1, ((0, 0), (0, 12)))
    b1p = jnp.pad(b1, ((0, 0), (0, 12)))
    wc = p["conv_W"] * (p["conv_g"] * s_bn)[None, :]
    bc = (p["conv_b"] * p["conv_g"] * s_bn + p["conv_be"])[None, :]
    wcp = jnp.pad(wc, ((0, 12), (0, 0)))
    wcat = jnp.concatenate([p["mean_W"], p["logv_W"]], axis=1)
    bcat = jnp.concatenate([p["mean_b"], p["logv_b"]])[None, :]
    wd0 = p["dec_W0"] * (p["dec_g0"] * s_bn)[None, :]
    bd0_t = (p["dec_b0"] * p["dec_g0"] * s_bn + p["dec_be0"])[:, None]
    wd0_t = wd0.T
    bcat_t = bcat.T
    wd1 = p["dec_W1"]
    bd1 = p["dec_b1"][None, :]
    cl = p["cluster"]
    csq_t = jnp.sum(cl * cl, axis=1)[:, None]

    src = adj[0]
    dst = adj[1]
    ar = jnp.arange(E_PAD - E, dtype=jnp.int32)
    src_pad = jnp.concatenate([src, ar % 997]).reshape(NW, CPW, CHUNK)
    dst_pad = jnp.concatenate([dst, N + ar % (N_PAD - N)]).reshape(NW, CPW, CHUNK)

    deg = _sc_degree(dst_pad)
    feat, xs1, dinv = _tc_encoder(x, deg.T[:N], w0, b0, w1p, b1p)
    p1 = _sc_propagate(xs1, src_pad, dst_pad, 32)
    xs2 = _tc_mid(p1, xs1, dinv, wcp, bc, wcat)
    p2 = _sc_propagate(xs2, src_pad, dst_pad, 16)
    zt, mut, lvt, de_feat, qt, f20t = _tc_tail(
        p2, xs2, dinv, feat, bcat_t, wd0_t, bd0_t, wd1, bd1, cl, csq_t)

    mu = mut.T
    return (zt.T, mu, lvt.T, de_feat, qt.T, f20t.T, mu)
